# baseline (speedup 1.0000x reference)
import functools

import jax
import jax.numpy as jnp
from jax import lax
from jax.experimental import pallas as pl
from jax.experimental.pallas import tpu as pltpu
from jax.experimental.pallas import tpu_sc as plsc

T = 2048
B = 32
M = 64
J = B * M
N = 200000

NC = 2
NS = 16
NW = NC * NS

SLAB_ROWS = 16
SLAB = SLAB_ROWS * J
NPASS = T // (NW * SLAB_ROWS)
CH = 10000
NCHUNK = N // CH
L = 16


def _sc_scatter_body(up_s, up_e, up_b, up_m, up_sc,
                     sd_s, sd_e, sd_b, sd_m, sd_sc,
                     dn_s, dn_e, dn_b, dn_m, dn_sc,
                     up_out, sd_out, dn_out, mask_out,
                     lslab, mslab, es, ee, eb, em, esc):
    wid = lax.axis_index("s") * NC + lax.axis_index("c")
    ones = jnp.ones((L,), jnp.float32)
    zeros = jnp.zeros((L,), jnp.float32)

    cats = [
        (up_s, up_e, up_b, up_m, up_sc, up_out),
        (sd_s, sd_e, sd_b, sd_m, sd_sc, sd_out),
        (dn_s, dn_e, dn_b, dn_m, dn_sc, dn_out),
    ]

    def zero_slab(slab):
        def zbody(k, _):
            slab[pl.ds(k * L, L)] = zeros
            return 0
        lax.fori_loop(0, SLAB // L, zbody, 0)

    for q in range(NPASS):
        lo = q * (NW * SLAB_ROWS) + wid * SLAB_ROWS
        hi = lo + SLAB_ROWS
        zero_slab(mslab)
        for (h_s, h_e, h_b, h_m, h_sc, h_out) in cats:
            zero_slab(lslab)

            def chunk_body(ci, _):
                base = ci * CH
                pltpu.sync_copy(h_s.at[pl.ds(base, CH)], es)
                pltpu.sync_copy(h_e.at[pl.ds(base, CH)], ee)
                pltpu.sync_copy(h_b.at[pl.ds(base, CH)], eb)
                pltpu.sync_copy(h_m.at[pl.ds(base, CH)], em)
                pltpu.sync_copy(h_sc.at[pl.ds(base, CH)], esc)

                def ev_body(i, _):
                    off = i * L
                    s = es[pl.ds(off, L)]
                    e = ee[pl.ds(off, L)]
                    ms = (s >= lo) & (s < hi)
                    me = (e >= lo) & (e < hi)

                    @pl.when(jnp.any(ms | me))
                    def _():
                        b = eb[pl.ds(off, L)]
                        m = em[pl.ds(off, L)]
                        sc = esc[pl.ds(off, L)]
                        j = b * M + m
                        idx_s = jnp.where(ms, (s - lo) * J + j, 0)
                        idx_e = jnp.where(me, (e - lo) * J + j, 0)
                        plsc.addupdate_scatter(lslab, [idx_s], ones, mask=ms)
                        plsc.addupdate_scatter(lslab, [idx_e], -ones, mask=me)
                        plsc.addupdate_scatter(mslab, [idx_s], sc, mask=ms)
                        plsc.addupdate_scatter(mslab, [idx_e], -sc, mask=me)

                    return 0

                lax.fori_loop(0, CH // L, ev_body, 0)
                return 0

            lax.fori_loop(0, NCHUNK, chunk_body, 0)
            pltpu.sync_copy(lslab, h_out.at[pl.ds(lo * J, SLAB)])
        pltpu.sync_copy(mslab, mask_out.at[pl.ds(lo * J, SLAB)])


@functools.partial(
    pl.kernel,
    out_type=[jax.ShapeDtypeStruct((T * J,), jnp.float32)] * 4,
    mesh=plsc.VectorSubcoreMesh(core_axis_name="c", subcore_axis_name="s",
                                num_cores=NC, num_subcores=NS),
    scratch_types=[
        pltpu.VMEM((SLAB,), jnp.float32),
        pltpu.VMEM((SLAB,), jnp.float32),
        pltpu.VMEM((CH,), jnp.int32),
        pltpu.VMEM((CH,), jnp.int32),
        pltpu.VMEM((CH,), jnp.int32),
        pltpu.VMEM((CH,), jnp.int32),
        pltpu.VMEM((CH,), jnp.float32),
    ],
    compiler_params=pltpu.CompilerParams(needs_layout_passes=False),
)
def _sc_scatter(*args):
    _sc_scatter_body(*args)


BT = 256


def _cumsum_body(u_in, s_in, d_in, m_in, u_out, s_out, d_out, m_out, carry):
    i = pl.program_id(0)

    @pl.when(i == 0)
    def _():
        carry[...] = jnp.zeros((4, J), jnp.float32)

    r = lax.broadcasted_iota(jnp.int32, (BT, BT), 0)
    c = lax.broadcasted_iota(jnp.int32, (BT, BT), 1)
    tri = (r >= c).astype(jnp.float32)

    for k, (xin, xout) in enumerate(
        [(u_in, u_out), (s_in, s_out), (d_in, d_out), (m_in, m_out)]
    ):
        x = xin[...]
        y = jax.lax.dot(tri, x, preferred_element_type=jnp.float32)
        y = y + carry[k:k + 1, :]
        xout[...] = y
        carry[k:k + 1, :] = y[BT - 1:BT, :]


_cumsum = pl.pallas_call(
    _cumsum_body,
    grid=(T // BT,),
    in_specs=[pl.BlockSpec((BT, J), lambda i: (i, 0))] * 4,
    out_specs=[pl.BlockSpec((BT, J), lambda i: (i, 0))] * 4,
    out_shape=[jax.ShapeDtypeStruct((T, J), jnp.float32)] * 4,
    scratch_shapes=[pltpu.VMEM((4, J), jnp.float32)],
)


def kernel(up_start, up_stop, up_batch, up_market, up_scores,
           side_start, side_stop, side_batch, side_market, side_scores,
           down_start, down_stop, down_batch, down_market, down_scores):
    ints = [x.astype(jnp.int32) for x in
            (up_start, up_stop, up_batch, up_market,
             side_start, side_stop, side_batch, side_market,
             down_start, down_stop, down_batch, down_market)]
    (u_s, u_e, u_b, u_m, s_s, s_e, s_b, s_m, d_s, d_e, d_b, d_m) = ints

    up_g, sd_g, dn_g, mk_g = _sc_scatter(
        u_s, u_e, u_b, u_m, up_scores,
        s_s, s_e, s_b, s_m, side_scores,
        d_s, d_e, d_b, d_m, down_scores)

    up_l, sd_l, dn_l, mask = _cumsum(
        up_g.reshape(T, J), sd_g.reshape(T, J),
        dn_g.reshape(T, J), mk_g.reshape(T, J))

    cats = jnp.stack([up_l.reshape(T, B, M),
                      sd_l.reshape(T, B, M),
                      dn_l.reshape(T, B, M)], axis=-1)
    return cats, mask.reshape(T, B, M)


# speedup vs baseline: 1.8815x; 1.8815x over previous
import functools

import jax
import jax.numpy as jnp
from jax import lax
from jax.experimental import pallas as pl
from jax.experimental.pallas import tpu as pltpu
from jax.experimental.pallas import tpu_sc as plsc

T = 2048
B = 32
M = 64
J = B * M
N = 200000

NC = 2
NS = 16
NW = NC * NS

SLAB_ROWS = 16
SLAB = SLAB_ROWS * J
NPASS = T // (NW * SLAB_ROWS)
CH = 10000
NCHUNK = N // CH
L = 16


def _sc_scatter_body(up_s, up_e, up_b, up_m, up_sc,
                     sd_s, sd_e, sd_b, sd_m, sd_sc,
                     dn_s, dn_e, dn_b, dn_m, dn_sc,
                     up_out, sd_out, dn_out, mask_out,
                     lslab, mslab, es, ee, eb, em, esc):
    wid = lax.axis_index("s") * NC + lax.axis_index("c")
    ones = jnp.ones((L,), jnp.float32)
    zeros = jnp.zeros((L,), jnp.float32)

    cats = [
        (up_s, up_e, up_b, up_m, up_sc, up_out),
        (sd_s, sd_e, sd_b, sd_m, sd_sc, sd_out),
        (dn_s, dn_e, dn_b, dn_m, dn_sc, dn_out),
    ]

    def zero_slab(slab):
        def zbody(k, _):
            slab[pl.ds(k * L, L)] = zeros
            return 0
        lax.fori_loop(0, SLAB // L, zbody, 0)

    for q in range(NPASS):
        lo = q * (NW * SLAB_ROWS) + wid * SLAB_ROWS
        hi = lo + SLAB_ROWS
        zero_slab(mslab)
        for (h_s, h_e, h_b, h_m, h_sc, h_out) in cats:
            zero_slab(lslab)

            def chunk_body(ci, _):
                base = ci * CH
                pltpu.sync_copy(h_s.at[pl.ds(base, CH)], es)
                pltpu.sync_copy(h_e.at[pl.ds(base, CH)], ee)
                pltpu.sync_copy(h_b.at[pl.ds(base, CH)], eb)
                pltpu.sync_copy(h_m.at[pl.ds(base, CH)], em)
                pltpu.sync_copy(h_sc.at[pl.ds(base, CH)], esc)

                def ev_body(i, _):
                    off = i * L
                    s = es[pl.ds(off, L)]
                    e = ee[pl.ds(off, L)]
                    ms = (s >= lo) & (s < hi)
                    me = (e >= lo) & (e < hi)
                    b = eb[pl.ds(off, L)]
                    m = em[pl.ds(off, L)]
                    sc = esc[pl.ds(off, L)]
                    j = b * M + m
                    idx_s = jnp.where(ms, (s - lo) * J + j, 0)
                    idx_e = jnp.where(me, (e - lo) * J + j, 0)
                    plsc.addupdate_scatter(lslab, [idx_s], ones, mask=ms)
                    plsc.addupdate_scatter(lslab, [idx_e], -ones, mask=me)
                    plsc.addupdate_scatter(mslab, [idx_s], sc, mask=ms)
                    plsc.addupdate_scatter(mslab, [idx_e], -sc, mask=me)
                    return 0

                lax.fori_loop(0, CH // L, ev_body, 0, unroll=8)
                return 0

            lax.fori_loop(0, NCHUNK, chunk_body, 0)
            pltpu.sync_copy(lslab, h_out.at[pl.ds(lo * J, SLAB)])
        pltpu.sync_copy(mslab, mask_out.at[pl.ds(lo * J, SLAB)])


@functools.partial(
    pl.kernel,
    out_type=[jax.ShapeDtypeStruct((T * J,), jnp.float32)] * 4,
    mesh=plsc.VectorSubcoreMesh(core_axis_name="c", subcore_axis_name="s",
                                num_cores=NC, num_subcores=NS),
    scratch_types=[
        pltpu.VMEM((SLAB,), jnp.float32),
        pltpu.VMEM((SLAB,), jnp.float32),
        pltpu.VMEM((CH,), jnp.int32),
        pltpu.VMEM((CH,), jnp.int32),
        pltpu.VMEM((CH,), jnp.int32),
        pltpu.VMEM((CH,), jnp.int32),
        pltpu.VMEM((CH,), jnp.float32),
    ],
    compiler_params=pltpu.CompilerParams(needs_layout_passes=False),
)
def _sc_scatter(*args):
    _sc_scatter_body(*args)


BT = 256


def _cumsum_body(u_in, s_in, d_in, m_in, u_out, s_out, d_out, m_out, carry):
    i = pl.program_id(0)

    @pl.when(i == 0)
    def _():
        carry[...] = jnp.zeros((4, J), jnp.float32)

    r = lax.broadcasted_iota(jnp.int32, (BT, BT), 0)
    c = lax.broadcasted_iota(jnp.int32, (BT, BT), 1)
    tri = (r >= c).astype(jnp.float32)

    for k, (xin, xout) in enumerate(
        [(u_in, u_out), (s_in, s_out), (d_in, d_out), (m_in, m_out)]
    ):
        x = xin[...]
        y = jax.lax.dot(tri, x, preferred_element_type=jnp.float32)
        y = y + carry[k:k + 1, :]
        xout[...] = y
        carry[k:k + 1, :] = y[BT - 1:BT, :]


_cumsum = pl.pallas_call(
    _cumsum_body,
    grid=(T // BT,),
    in_specs=[pl.BlockSpec((BT, J), lambda i: (i, 0))] * 4,
    out_specs=[pl.BlockSpec((BT, J), lambda i: (i, 0))] * 4,
    out_shape=[jax.ShapeDtypeStruct((T, J), jnp.float32)] * 4,
    scratch_shapes=[pltpu.VMEM((4, J), jnp.float32)],
)


def kernel(up_start, up_stop, up_batch, up_market, up_scores,
           side_start, side_stop, side_batch, side_market, side_scores,
           down_start, down_stop, down_batch, down_market, down_scores):
    ints = [x.astype(jnp.int32) for x in
            (up_start, up_stop, up_batch, up_market,
             side_start, side_stop, side_batch, side_market,
             down_start, down_stop, down_batch, down_market)]
    (u_s, u_e, u_b, u_m, s_s, s_e, s_b, s_m, d_s, d_e, d_b, d_m) = ints

    up_g, sd_g, dn_g, mk_g = _sc_scatter(
        u_s, u_e, u_b, u_m, up_scores,
        s_s, s_e, s_b, s_m, side_scores,
        d_s, d_e, d_b, d_m, down_scores)

    up_l, sd_l, dn_l, mask = _cumsum(
        up_g.reshape(T, J), sd_g.reshape(T, J),
        dn_g.reshape(T, J), mk_g.reshape(T, J))

    cats = jnp.stack([up_l.reshape(T, B, M),
                      sd_l.reshape(T, B, M),
                      dn_l.reshape(T, B, M)], axis=-1)
    return cats, mask.reshape(T, B, M)
